# Initial kernel scaffold; baseline (speedup 1.0000x reference)
import jax
import jax.numpy as jnp
from jax.experimental import pallas as pl


def kernel(x, Wg, Wgate, Wup, Wdown):
    raise NotImplementedError("write your pallas kernel here")


# speedup vs baseline: 1.0787x; 1.0787x over previous
import functools

import jax
import jax.numpy as jnp
from jax import lax
from jax.experimental import pallas as pl
from jax.experimental.pallas import tpu as pltpu

_TOPK = 2
_BM = 256
_BFF = 1408
_INTERPRET = False


def _gate_body(x_ref, wg_ref, i1_ref, i2_ref, w1_ref, w2_ref):
    x = x_ref[...]
    logits = lax.dot_general(x, wg_ref[...], (((1,), (1,)), ((), ())),
                             preferred_element_type=jnp.float32)
    m = jnp.max(logits, axis=-1, keepdims=True)
    ex = jnp.exp(logits - m)
    p = ex / jnp.sum(ex, axis=-1, keepdims=True)
    e_num = p.shape[-1]
    idxs = lax.broadcasted_iota(jnp.int32, p.shape, 1)
    m1 = jnp.max(p, axis=-1, keepdims=True)
    a1 = jnp.min(jnp.where(p == m1, idxs, e_num), axis=-1, keepdims=True)
    p2 = jnp.where(idxs == a1, -1.0, p)
    m2 = jnp.max(p2, axis=-1, keepdims=True)
    a2 = jnp.min(jnp.where(p2 == m2, idxs, e_num), axis=-1, keepdims=True)
    s = m1 + m2 + 1e-20
    i1_ref[...] = a1
    i2_ref[...] = a2
    w1_ref[...] = m1 / s
    w2_ref[...] = m2 / s


def _gate(xf, Wg):
    t = xf.shape[0]
    return pl.pallas_call(
        _gate_body,
        out_shape=[
            jax.ShapeDtypeStruct((t, 1), jnp.int32),
            jax.ShapeDtypeStruct((t, 1), jnp.int32),
            jax.ShapeDtypeStruct((t, 1), jnp.float32),
            jax.ShapeDtypeStruct((t, 1), jnp.float32),
        ],
        interpret=_INTERPRET,
    )(xf, Wg)


def _gmm_body(ff, e_ref, b_ref, rs_ref, re_ref, f_ref,
              xs_ref, wg_ref, wu_ref, wd_ref, ws_ref, out_ref):
    i = pl.program_id(0)
    j = pl.program_id(1)

    @pl.when(jnp.logical_and(j == 0, f_ref[i] == 1))
    def _():
        out_ref[...] = jnp.zeros_like(out_ref)

    rows = b_ref[i] * _BM + lax.broadcasted_iota(jnp.int32, (_BM, 1), 0)
    wv = jnp.where((rows >= rs_ref[i]) & (rows < re_ref[i]), ws_ref[...], 0.0)
    x = xs_ref[...]
    g = jnp.dot(x, wg_ref[0], preferred_element_type=jnp.float32)
    u = jnp.dot(x, wu_ref[0], preferred_element_type=jnp.float32)
    h = (g * jax.nn.sigmoid(g)) * u * wv
    cols = j * _BFF + lax.broadcasted_iota(jnp.int32, (1, _BFF), 1)
    h = jnp.where(cols < ff, h, 0.0)
    rows_ff = j * _BFF + lax.broadcasted_iota(jnp.int32, (_BFF, 1), 0)
    wd = jnp.where(rows_ff < ff, wd_ref[0], 0.0)
    out_ref[...] += jnp.dot(h, wd, preferred_element_type=jnp.float32)


def _gmm(xs, ws, Wgate, Wup, Wdown, wi_e, wi_b, wi_rs, wi_re, wi_first, nw):
    m, h = xs.shape
    ff = Wgate.shape[2]
    nff = -(-ff // _BFF)
    grid_spec = pltpu.PrefetchScalarGridSpec(
        num_scalar_prefetch=5,
        grid=(nw, nff),
        in_specs=[
            pl.BlockSpec((_BM, h), lambda i, j, e, b, rs, re, f: (b[i], 0)),
            pl.BlockSpec((1, h, _BFF), lambda i, j, e, b, rs, re, f: (e[i], 0, j)),
            pl.BlockSpec((1, h, _BFF), lambda i, j, e, b, rs, re, f: (e[i], 0, j)),
            pl.BlockSpec((1, _BFF, h), lambda i, j, e, b, rs, re, f: (e[i], j, 0)),
            pl.BlockSpec((_BM, 1), lambda i, j, e, b, rs, re, f: (b[i], 0)),
        ],
        out_specs=pl.BlockSpec((_BM, h), lambda i, j, e, b, rs, re, f: (b[i], 0)),
    )
    return pl.pallas_call(
        functools.partial(_gmm_body, ff),
        grid_spec=grid_spec,
        out_shape=jax.ShapeDtypeStruct((m, h), jnp.float32),
        compiler_params=pltpu.CompilerParams(
            dimension_semantics=("arbitrary", "arbitrary")),
        interpret=_INTERPRET,
    )(wi_e, wi_b, wi_rs, wi_re, wi_first,
      xs, Wgate, Wup, Wdown, ws.reshape(m, 1))


def kernel(x, Wg, Wgate, Wup, Wdown):
    b, s, h = x.shape
    e_num = Wg.shape[0]
    xf = x.reshape(-1, h)
    t = xf.shape[0]
    m = t * _TOPK
    nb = m // _BM
    nw = nb + e_num - 1

    a1, a2, w1, w2 = _gate(xf, Wg)
    eid = jnp.concatenate([a1, a2], axis=1).reshape(-1)
    w_flat = jnp.concatenate([w1, w2], axis=1).reshape(-1)

    order = jnp.argsort(eid, stable=True)
    tok_ids = order // _TOPK
    ip = jnp.argsort(order)
    p0, p1 = ip[0::2], ip[1::2]

    counts = jnp.sum(eid[:, None] == jnp.arange(e_num)[None, :], axis=0,
                     dtype=jnp.int32)
    off = jnp.concatenate([jnp.zeros((1,), jnp.int32), jnp.cumsum(counts)])
    starts_e, ends_e = off[:e_num], off[1:]
    b_start = jnp.arange(nb, dtype=jnp.int32) * _BM
    overlap = ((starts_e[None, :] < (b_start + _BM)[:, None])
               & (ends_e[None, :] > b_start[:, None])
               & (counts[None, :] > 0))
    sel = jnp.nonzero(overlap.reshape(-1), size=nw,
                      fill_value=nb * e_num - 1)[0]
    nvalid = jnp.sum(overlap)
    dummy = jnp.arange(nw) >= nvalid
    wi_b = (sel // e_num).astype(jnp.int32)
    wi_e = (sel % e_num).astype(jnp.int32)
    wi_rs = jnp.where(dummy, 0, jnp.maximum(starts_e[wi_e], wi_b * _BM))
    wi_re = jnp.where(dummy, 0, jnp.minimum(ends_e[wi_e], (wi_b + 1) * _BM))
    wi_first = jnp.concatenate(
        [jnp.ones((1,), jnp.int32),
         (wi_b[1:] != wi_b[:-1]).astype(jnp.int32)])

    xs = xf[tok_ids]
    ws = w_flat[order]

    ys = _gmm(xs, ws, Wgate, Wup, Wdown, wi_e, wi_b,
              wi_rs.astype(jnp.int32), wi_re.astype(jnp.int32), wi_first, nw)

    y = ys[p0] + ys[p1]
    return y.reshape(b, s, h)


# speedup vs baseline: 1.1620x; 1.0772x over previous
import functools

import jax
import jax.numpy as jnp
from jax import lax
from jax.experimental import pallas as pl
from jax.experimental.pallas import tpu as pltpu
from jax.experimental.pallas import tpu_sc as plsc

_TOPK = 2
_BM = 256
_BFF = 1408
_NC = 2
_NS = 16
_NWRK = _NC * _NS


def _gate_body(x_ref, wg_ref, i1_ref, i2_ref, w1_ref, w2_ref):
    x = x_ref[...]
    logits = lax.dot_general(x, wg_ref[...], (((1,), (1,)), ((), ())),
                             preferred_element_type=jnp.float32)
    m = jnp.max(logits, axis=-1, keepdims=True)
    ex = jnp.exp(logits - m)
    p = ex / jnp.sum(ex, axis=-1, keepdims=True)
    e_num = p.shape[-1]
    idxs = lax.broadcasted_iota(jnp.int32, p.shape, 1)
    m1 = jnp.max(p, axis=-1, keepdims=True)
    a1 = jnp.min(jnp.where(p == m1, idxs, e_num), axis=-1, keepdims=True)
    p2 = jnp.where(idxs == a1, -1.0, p)
    m2 = jnp.max(p2, axis=-1, keepdims=True)
    a2 = jnp.min(jnp.where(p2 == m2, idxs, e_num), axis=-1, keepdims=True)
    s = m1 + m2 + 1e-20
    i1_ref[...] = a1
    i2_ref[...] = a2
    w1_ref[...] = jnp.broadcast_to(m1 / s, w1_ref.shape)
    w2_ref[...] = jnp.broadcast_to(m2 / s, w2_ref.shape)


def _gate(xf, Wg):
    t = xf.shape[0]
    return pl.pallas_call(
        _gate_body,
        out_shape=[
            jax.ShapeDtypeStruct((t, 1), jnp.int32),
            jax.ShapeDtypeStruct((t, 1), jnp.int32),
            jax.ShapeDtypeStruct((t, 16), jnp.float32),
            jax.ShapeDtypeStruct((t, 16), jnp.float32),
        ],
    )(xf, Wg)


def _dispatch(xf, tok_ids):
    t, h = xf.shape
    m = tok_ids.shape[0]
    spw = m // _NWRK
    nch = 4
    ch = spw // nch
    mesh = plsc.VectorSubcoreMesh(core_axis_name="c", subcore_axis_name="s")

    @functools.partial(
        pl.kernel, mesh=mesh,
        out_type=jax.ShapeDtypeStruct((m, h), jnp.float32),
        scratch_types=(
            [pltpu.VMEM((ch,), jnp.int32)] * nch
            + [pltpu.VMEM((ch, h), jnp.float32)] * 2
            + [pltpu.SemaphoreType.DMA] * 2
        ),
    )
    def k(xf_hbm, tok_hbm, xs_hbm, *refs):
        toks = refs[:nch]
        bufs = refs[nch:nch + 2]
        sems = refs[nch + 2:]
        wid = lax.axis_index("s") * _NC + lax.axis_index("c")
        base = wid * spw
        for c in range(nch):
            pltpu.sync_copy(tok_hbm.at[pl.ds(base + c * ch, ch)], toks[c])
        cps = [pltpu.async_copy(xf_hbm.at[toks[0]], bufs[0], sems[0]),
               pltpu.async_copy(xf_hbm.at[toks[1]], bufs[1], sems[1]),
               None, None]
        for c in range(nch):
            cps[c].wait()
            pltpu.sync_copy(bufs[c % 2], xs_hbm.at[pl.ds(base + c * ch, ch)])
            if c + 2 < nch:
                cps[c + 2] = pltpu.async_copy(
                    xf_hbm.at[toks[c + 2]], bufs[c % 2], sems[c % 2])

    return k(xf, tok_ids)


def _combine(ys, p0, p1, w1b, w2b):
    m, h = ys.shape
    t = p0.shape[0]
    tpw = t // _NWRK
    half = tpw // 2
    mesh = plsc.VectorSubcoreMesh(core_axis_name="c", subcore_axis_name="s")

    @functools.partial(
        pl.kernel, mesh=mesh,
        out_type=jax.ShapeDtypeStruct((t, h), jnp.float32),
        scratch_types=[
            pltpu.VMEM((half,), jnp.int32),
            pltpu.VMEM((half,), jnp.int32),
            pltpu.VMEM((half, 16), jnp.float32),
            pltpu.VMEM((half, 16), jnp.float32),
            pltpu.VMEM((half, h), jnp.float32),
            pltpu.VMEM((half, h), jnp.float32),
            pltpu.SemaphoreType.DMA,
            pltpu.SemaphoreType.DMA,
        ],
    )
    def k(ys_hbm, p0_hbm, p1_hbm, w1_hbm, w2_hbm, y_hbm,
          pa, pb, w1_v, w2_v, a_v, b_v, sem0, sem1):
        wid = lax.axis_index("s") * _NC + lax.axis_index("c")
        base = wid * tpw

        def do_half(off):
            pltpu.sync_copy(p0_hbm.at[pl.ds(base + off, half)], pa)
            pltpu.sync_copy(p1_hbm.at[pl.ds(base + off, half)], pb)
            pltpu.sync_copy(w1_hbm.at[pl.ds(base + off, half)], w1_v)
            pltpu.sync_copy(w2_hbm.at[pl.ds(base + off, half)], w2_v)
            cp0 = pltpu.async_copy(ys_hbm.at[pa], a_v, sem0)
            cp1 = pltpu.async_copy(ys_hbm.at[pb], b_v, sem1)
            cp0.wait()
            cp1.wait()

            def row(r, carry):
                wa = w1_v[r, :]
                wb = w2_v[r, :]
                for c in range(h // 16):
                    sl = pl.ds(c * 16, 16)
                    a_v[r, sl] = a_v[r, sl] * wa + b_v[r, sl] * wb
                return carry

            lax.fori_loop(0, half, row, 0)
            pltpu.sync_copy(a_v, y_hbm.at[pl.ds(base + off, half)])

        do_half(0)
        do_half(half)

    return k(ys, p0, p1, w1b, w2b)


def _gmm_body(ff, e_ref, b_ref, rs_ref, re_ref, f_ref,
              xs_ref, wg_ref, wu_ref, wd_ref, out_ref):
    i = pl.program_id(0)
    j = pl.program_id(1)

    @pl.when(jnp.logical_and(j == 0, f_ref[i] == 1))
    def _():
        out_ref[...] = jnp.zeros_like(out_ref)

    rows = b_ref[i] * _BM + lax.broadcasted_iota(jnp.int32, (_BM, 1), 0)
    inrange = (rows >= rs_ref[i]) & (rows < re_ref[i])
    x = jnp.where(inrange, xs_ref[...], 0.0)
    g = jnp.dot(x, wg_ref[0], preferred_element_type=jnp.float32)
    u = jnp.dot(x, wu_ref[0], preferred_element_type=jnp.float32)
    h = (g * jax.nn.sigmoid(g)) * u
    cols = j * _BFF + lax.broadcasted_iota(jnp.int32, (1, _BFF), 1)
    h = jnp.where(cols < ff, h, 0.0)
    rows_ff = j * _BFF + lax.broadcasted_iota(jnp.int32, (_BFF, 1), 0)
    wd = jnp.where(rows_ff < ff, wd_ref[0], 0.0)
    out_ref[...] += jnp.dot(h, wd, preferred_element_type=jnp.float32)


def _gmm(xs, Wgate, Wup, Wdown, wi_e, wi_b, wi_rs, wi_re, wi_first, nw):
    m, h = xs.shape
    ff = Wgate.shape[2]
    nff = -(-ff // _BFF)
    grid_spec = pltpu.PrefetchScalarGridSpec(
        num_scalar_prefetch=5,
        grid=(nw, nff),
        in_specs=[
            pl.BlockSpec((_BM, h), lambda i, j, e, b, rs, re, f: (b[i], 0)),
            pl.BlockSpec((1, h, _BFF), lambda i, j, e, b, rs, re, f: (e[i], 0, j)),
            pl.BlockSpec((1, h, _BFF), lambda i, j, e, b, rs, re, f: (e[i], 0, j)),
            pl.BlockSpec((1, _BFF, h), lambda i, j, e, b, rs, re, f: (e[i], j, 0)),
        ],
        out_specs=pl.BlockSpec((_BM, h), lambda i, j, e, b, rs, re, f: (b[i], 0)),
    )
    return pl.pallas_call(
        functools.partial(_gmm_body, ff),
        grid_spec=grid_spec,
        out_shape=jax.ShapeDtypeStruct((m, h), jnp.float32),
        compiler_params=pltpu.CompilerParams(
            dimension_semantics=("arbitrary", "arbitrary")),
    )(wi_e, wi_b, wi_rs, wi_re, wi_first, xs, Wgate, Wup, Wdown)


def kernel(x, Wg, Wgate, Wup, Wdown):
    b, s, h = x.shape
    e_num = Wg.shape[0]
    xf = x.reshape(-1, h)
    t = xf.shape[0]
    m = t * _TOPK
    nb = m // _BM
    nw = nb + e_num - 1

    a1, a2, w1b, w2b = _gate(xf, Wg)
    eid = jnp.concatenate([a1, a2], axis=1).reshape(-1)

    order = jnp.argsort(eid, stable=True)
    tok_ids = (order // _TOPK).astype(jnp.int32)
    ip = jnp.argsort(order)
    p0, p1 = ip[0::2], ip[1::2]

    counts = jnp.sum(eid[:, None] == jnp.arange(e_num)[None, :], axis=0,
                     dtype=jnp.int32)
    off = jnp.concatenate([jnp.zeros((1,), jnp.int32), jnp.cumsum(counts)])
    starts_e, ends_e = off[:e_num], off[1:]
    b_start = jnp.arange(nb, dtype=jnp.int32) * _BM
    overlap = ((starts_e[None, :] < (b_start + _BM)[:, None])
               & (ends_e[None, :] > b_start[:, None])
               & (counts[None, :] > 0))
    sel = jnp.nonzero(overlap.reshape(-1), size=nw,
                      fill_value=nb * e_num - 1)[0]
    nvalid = jnp.sum(overlap)
    dummy = jnp.arange(nw) >= nvalid
    wi_b = (sel // e_num).astype(jnp.int32)
    wi_e = (sel % e_num).astype(jnp.int32)
    wi_rs = jnp.where(dummy, 0, jnp.maximum(starts_e[wi_e], wi_b * _BM))
    wi_re = jnp.where(dummy, 0, jnp.minimum(ends_e[wi_e], (wi_b + 1) * _BM))
    wi_first = jnp.concatenate(
        [jnp.ones((1,), jnp.int32),
         (wi_b[1:] != wi_b[:-1]).astype(jnp.int32)])

    xs = _dispatch(xf, tok_ids)

    ys = _gmm(xs, Wgate, Wup, Wdown, wi_e, wi_b,
              wi_rs.astype(jnp.int32), wi_re.astype(jnp.int32), wi_first, nw)

    y = _combine(ys, p0.astype(jnp.int32), p1.astype(jnp.int32), w1b, w2b)
    return y.reshape(b, s, h)
